# Initial kernel scaffold; baseline (speedup 1.0000x reference)
import jax
import jax.numpy as jnp
from jax.experimental import pallas as pl


def kernel(weight_probs, weight_indices, bias_probs, bias_indices, weight_bank, bias_bank):
    raise NotImplementedError("write your pallas kernel here")


# speedup vs baseline: 41.4921x; 41.4921x over previous
import functools

import jax
import jax.numpy as jnp
from jax.experimental import pallas as pl
from jax.experimental.pallas import tpu as pltpu

INPUT_DIM = 768
OUTPUT_DIM = 768
NUM_EXPERTS = 16
TOP_K = 2
BATCH = 64

BI = 16


def _weight_body(wp0_ref, wp1_ref, wi0_ref, wi1_ref, bank_ref, out_ref):
    bank = bank_ref[...]
    e_iota = jax.lax.broadcasted_iota(jnp.int32, (BI, BATCH, NUM_EXPERTS), 2)
    s = jnp.where(wi0_ref[...][:, :, None] == e_iota, wp0_ref[...][:, :, None], 0.0)
    s = s + jnp.where(wi1_ref[...][:, :, None] == e_iota, wp1_ref[...][:, :, None], 0.0)
    res = jax.lax.dot_general(
        s, bank,
        dimension_numbers=(((2,), (1,)), ((0,), (0,))),
        preferred_element_type=jnp.float32)
    out_ref[...] = jnp.transpose(res, (1, 0, 2))


def _bias_body(bp0_ref, bp1_ref, bi0_ref, bi1_ref, bank_ref, out_ref):
    bi0 = bi0_ref[...]
    bi1 = bi1_ref[...]
    val0 = jnp.zeros((OUTPUT_DIM, BATCH), jnp.float32)
    val1 = jnp.zeros((OUTPUT_DIM, BATCH), jnp.float32)
    for e in range(NUM_EXPERTS):
        col = bank_ref[:, e][:, None]
        val0 = val0 + jnp.where(bi0 == e, col, 0.0)
        val1 = val1 + jnp.where(bi1 == e, col, 0.0)
    res = bp0_ref[...] * val0 + bp1_ref[...] * val1
    out_ref[...] = res.T


@jax.jit
def kernel(weight_probs, weight_indices, bias_probs, bias_indices,
           weight_bank, bias_bank):
    wp0, wp1 = weight_probs[:, :, 0], weight_probs[:, :, 1]
    wi0, wi1 = weight_indices[:, :, 0], weight_indices[:, :, 1]
    bp0, bp1 = bias_probs[:, :, 0], bias_probs[:, :, 1]
    bi0, bi1 = bias_indices[:, :, 0], bias_indices[:, :, 1]

    nblk = INPUT_DIM // BI
    dxb = pl.BlockSpec((BI, BATCH), lambda i: (i, 0))
    weight_mixture = pl.pallas_call(
        _weight_body,
        grid=(nblk,),
        in_specs=[
            dxb, dxb, dxb, dxb,
            pl.BlockSpec((BI, NUM_EXPERTS, OUTPUT_DIM), lambda i: (i, 0, 0)),
        ],
        out_specs=pl.BlockSpec((BATCH, BI, OUTPUT_DIM), lambda i: (0, i, 0)),
        out_shape=jax.ShapeDtypeStruct((BATCH, INPUT_DIM, OUTPUT_DIM),
                                       jnp.float32),
    )(wp0, wp1, wi0, wi1, weight_bank)

    bias_mixture = pl.pallas_call(
        _bias_body,
        out_shape=jax.ShapeDtypeStruct((BATCH, OUTPUT_DIM), jnp.float32),
    )(bp0, bp1, bi0, bi1, bias_bank)

    return weight_mixture, bias_mixture
